# baseline (speedup 1.0000x reference)
import functools

import jax
import jax.numpy as jnp
from jax.experimental import pallas as pl
from jax.experimental.pallas import tpu as pltpu

ALPHA_C = 3.0
K_C = 30
_LANE = 128


def _bisect_unrolled(count_ge, k, lo0, hi0, steps):
    lo, hi = lo0, hi0
    for _ in range(steps):
        mid = lo + ((hi - lo) >> 1)
        ok = count_ge(mid) >= k
        lo = jnp.where(ok, mid, lo)
        hi = jnp.where(ok, hi, mid)
    return lo


def _search_while(count_ge, k, lo0, hi0):
    cl0 = count_ge(lo0)
    ch0 = jnp.zeros_like(cl0)

    def cond(carry):
        lo, hi, cl, ch = carry
        return jnp.max(hi - lo) > 1

    def probe(carry, mid):
        lo, hi, cl, ch = carry
        cm = count_ge(mid)
        ok = cm >= k
        return (jnp.where(ok, mid, lo), jnp.where(ok, hi, mid),
                jnp.where(ok, cm, cl), jnp.where(ok, ch, cm))

    def body(carry):
        lo, hi, cl, ch = carry
        span = hi - lo
        frac = (cl - k + 1).astype(jnp.float32) / (cl - ch + 1).astype(
            jnp.float32)
        off = (span.astype(jnp.float32) * frac).astype(jnp.int32)
        carry = probe(carry, lo + jnp.clip(off, 1, span - 1))
        lo, hi, cl, ch = carry
        return probe(carry, lo + ((hi - lo) >> 1))

    lo, _, _, _ = jax.lax.while_loop(cond, body, (lo0, hi0, cl0, ch0))
    return lo


def _body(a_ref, o_ref, u_ref, *, n_cols, k):
    x = a_ref[...]
    adj = jnp.tanh(ALPHA_C * x)
    o_ref[...] = adj
    u = jax.lax.bitcast_convert_type(jnp.abs(adj), jnp.int32)
    br = x.shape[0]
    n_pad = u_ref.shape[1]
    n_ch = n_pad // _LANE
    u_ref[:, n_cols:n_pad] = jnp.zeros((br, n_pad - n_cols), jnp.int32)
    u_ref[:, 0:n_cols] = u

    mx = u_ref[:, 0:_LANE]
    for i in range(1, n_ch):
        mx = jnp.maximum(mx, u_ref[:, i * _LANE:(i + 1) * _LANE])
    rowmax = jnp.max(mx, axis=1, keepdims=True)
    rowmin = jnp.min(mx, axis=1, keepdims=True)

    def count_mx(t):
        return jnp.sum((mx >= t).astype(jnp.int32), axis=1, keepdims=True)

    m_k = _bisect_unrolled(count_mx, k, rowmin, rowmax + 1, steps=31)

    def count_u(t):
        return jnp.sum((u_ref[...] >= t).astype(jnp.int32), axis=1,
                       keepdims=True)

    t_bits = _search_while(count_u, k, m_k, rowmax + 1)

    uu = u_ref[...]
    count_gt = jnp.sum((uu > t_bits).astype(jnp.int32), axis=1,
                       keepdims=True)
    r = k - count_gt

    acc = jnp.zeros((br, 1), jnp.int32)
    b_vec = jnp.zeros((br, _LANE), jnp.int32)
    r_in = jnp.zeros((br, 1), jnp.int32)
    j_base = jnp.zeros((br, 1), jnp.int32)
    for i in range(n_ch):
        tie_i = (u_ref[:, i * _LANE:(i + 1) * _LANE] ==
                 t_bits).astype(jnp.int32)
        c_i = jnp.sum(tie_i, axis=1, keepdims=True)
        isb = (acc < r) & (acc + c_i >= r)
        b_vec = b_vec + jnp.where(isb, tie_i, 0)
        r_in = r_in + jnp.where(isb, r - acc, 0)
        j_base = j_base + jnp.where(isb, i * _LANE, 0)
        acc = acc + c_i

    ii = jax.lax.broadcasted_iota(jnp.int32, (_LANE, _LANE), 0)
    jj = jax.lax.broadcasted_iota(jnp.int32, (_LANE, _LANE), 1)
    tri = (ii <= jj).astype(jnp.float32)
    cum_b = jnp.dot(b_vec.astype(jnp.float32), tri,
                    preferred_element_type=jnp.float32).astype(jnp.int32)
    l_in = jnp.sum((cum_b < r_in).astype(jnp.int32), axis=1, keepdims=True)
    j_cut = j_base + l_in

    u_real = uu[:, 0:n_cols]
    iota = jax.lax.broadcasted_iota(jnp.int32, (br, n_cols), 1)
    keep = (u_real > t_bits) | ((u_real == t_bits) & (iota <= j_cut))
    o_ref[...] = jnp.where(keep, o_ref[...], 0.0)


def kernel(idx, A):
    n, n_cols = A.shape
    del idx
    br = next(b for b in (40, 80, 16, 8, 1) if n % b == 0)
    n_pad = ((n_cols + _LANE - 1) // _LANE) * _LANE
    if n_pad % 256:
        n_pad += _LANE
    body = functools.partial(_body, n_cols=n_cols, k=K_C)
    return pl.pallas_call(
        body,
        grid=(n // br,),
        in_specs=[pl.BlockSpec((br, n_cols), lambda i: (i, 0))],
        out_specs=pl.BlockSpec((br, n_cols), lambda i: (i, 0)),
        out_shape=jax.ShapeDtypeStruct((n, n_cols), jnp.float32),
        scratch_shapes=[
            pltpu.VMEM((br, n_pad), jnp.int32),
        ],
    )(A)


# speedup vs baseline: 1.6863x; 1.6863x over previous
import functools

import jax
import jax.numpy as jnp
from jax.experimental import pallas as pl
from jax.experimental.pallas import tpu as pltpu

ALPHA_C = 3.0
K_C = 30
_LANE = 128


def _bisect_unrolled(count_ge, k, lo0, hi0, steps):
    lo, hi = lo0, hi0
    for _ in range(steps):
        mid = lo + ((hi - lo) >> 1)
        ok = count_ge(mid) >= k
        lo = jnp.where(ok, mid, lo)
        hi = jnp.where(ok, hi, mid)
    return lo


def _search_while(count_ge, k, lo0, hi0):
    cl0 = count_ge(lo0)
    ch0 = jnp.zeros_like(cl0)

    def cond(carry):
        lo, hi, cl, ch = carry
        return jnp.max(hi - lo) > 1

    def probe(carry, mid):
        lo, hi, cl, ch = carry
        cm = count_ge(mid)
        ok = cm >= k
        return (jnp.where(ok, mid, lo), jnp.where(ok, hi, mid),
                jnp.where(ok, cm, cl), jnp.where(ok, ch, cm))

    def body(carry):
        lo, hi, cl, ch = carry
        span = hi - lo
        frac = (cl - k + 1).astype(jnp.float32) / (cl - ch + 1).astype(
            jnp.float32)
        off = (span.astype(jnp.float32) * frac).astype(jnp.int32)
        carry = probe(carry, lo + jnp.clip(off, 1, span - 1))
        lo, hi, cl, ch = carry
        return probe(carry, lo + ((hi - lo) >> 1))

    lo, _, _, _ = jax.lax.while_loop(cond, body, (lo0, hi0, cl0, ch0))
    return lo


def _body(a_ref, o_ref, u_ref, *, n_cols, k):
    x = a_ref[...]
    adj = jnp.tanh(ALPHA_C * x)
    o_ref[...] = adj
    u = jax.lax.bitcast_convert_type(jnp.abs(adj), jnp.int32)
    br = x.shape[0]
    n_pad = u_ref.shape[1]
    n_ch = n_pad // _LANE
    u_ref[:, n_cols:n_pad] = jnp.zeros((br, n_pad - n_cols), jnp.int32)
    u_ref[:, 0:n_cols] = u

    mx = u_ref[:, 0:_LANE]
    for i in range(1, n_ch):
        mx = jnp.maximum(mx, u_ref[:, i * _LANE:(i + 1) * _LANE])
    rowmax = jnp.max(mx, axis=1, keepdims=True)
    rowmin = jnp.min(mx, axis=1, keepdims=True)

    def count_mx(t):
        return jnp.sum((mx >= t).astype(jnp.int32), axis=1, keepdims=True)

    m_k = _bisect_unrolled(count_mx, k, rowmin, rowmax + 1, steps=31)

    def count_u(t):
        return jnp.sum((u_ref[...] >= t).astype(jnp.int32), axis=1,
                       keepdims=True)

    t_bits = _search_while(count_u, k, m_k, rowmax + 1)

    uu = u_ref[...]
    count_gt = jnp.sum((uu > t_bits).astype(jnp.int32), axis=1,
                       keepdims=True)
    r = k - count_gt

    acc = jnp.zeros((br, 1), jnp.int32)
    b_vec = jnp.zeros((br, _LANE), jnp.int32)
    r_in = jnp.zeros((br, 1), jnp.int32)
    j_base = jnp.zeros((br, 1), jnp.int32)
    for i in range(n_ch):
        tie_i = (u_ref[:, i * _LANE:(i + 1) * _LANE] ==
                 t_bits).astype(jnp.int32)
        c_i = jnp.sum(tie_i, axis=1, keepdims=True)
        isb = (acc < r) & (acc + c_i >= r)
        b_vec = b_vec + jnp.where(isb, tie_i, 0)
        r_in = r_in + jnp.where(isb, r - acc, 0)
        j_base = j_base + jnp.where(isb, i * _LANE, 0)
        acc = acc + c_i

    ii = jax.lax.broadcasted_iota(jnp.int32, (_LANE, _LANE), 0)
    jj = jax.lax.broadcasted_iota(jnp.int32, (_LANE, _LANE), 1)
    tri = (ii <= jj).astype(jnp.float32)
    cum_b = jnp.dot(b_vec.astype(jnp.float32), tri,
                    preferred_element_type=jnp.float32).astype(jnp.int32)
    l_in = jnp.sum((cum_b < r_in).astype(jnp.int32), axis=1, keepdims=True)
    j_cut = j_base + l_in

    u_real = uu[:, 0:n_cols]
    iota = jax.lax.broadcasted_iota(jnp.int32, (br, n_cols), 1)
    keep = (u_real > t_bits) | ((u_real == t_bits) & (iota <= j_cut))
    o_ref[...] = jnp.where(keep, o_ref[...], 0.0)


def kernel(idx, A):
    n, n_cols = A.shape
    del idx
    br = next(b for b in (200, 80, 40, 16, 8, 1) if n % b == 0)
    n_pad = ((n_cols + _LANE - 1) // _LANE) * _LANE
    if n_pad % 256:
        n_pad += _LANE
    body = functools.partial(_body, n_cols=n_cols, k=K_C)
    return pl.pallas_call(
        body,
        grid=(n // br,),
        in_specs=[pl.BlockSpec((br, n_cols), lambda i: (i, 0))],
        out_specs=pl.BlockSpec((br, n_cols), lambda i: (i, 0)),
        out_shape=jax.ShapeDtypeStruct((n, n_cols), jnp.float32),
        scratch_shapes=[
            pltpu.VMEM((br, n_pad), jnp.int32),
        ],
    )(A)
